# Initial kernel scaffold; baseline (speedup 1.0000x reference)
import jax
import jax.numpy as jnp
from jax.experimental import pallas as pl


def kernel(x, historical_data, W1, b1, W2, b2, gamma, beta):
    raise NotImplementedError("write your pallas kernel here")


# speedup vs baseline: 2.4243x; 2.4243x over previous
import functools

import jax
import jax.numpy as jnp
from jax import lax
from jax.experimental import pallas as pl
from jax.experimental.pallas import tpu as pltpu
from jax.experimental.pallas import tpu_sc as plsc

B = 1024
D = 16
K_HIST = 100000
K_NEIGH = 5
KC = 2048
K_PAD = 100352
NK = K_PAD // KC
NEG = -3.0e38
BIGI = 0x3FFFFFFF
EPS = 1e-8


def _topk_body(x_ref, h_ref, outv_ref, outi_ref, vals_s, idx_s):
    i = pl.program_id(0)

    @pl.when(i == 0)
    def _init():
        vals_s[...] = jnp.full((B, 128), NEG, jnp.float32)
        idx_s[...] = jnp.full((B, 128), BIGI, jnp.int32)

    x = x_ref[...]
    xn = jnp.maximum(jnp.sqrt(jnp.sum(x * x, axis=1, keepdims=True)), EPS)
    h = h_ref[...]
    hn = jnp.maximum(jnp.sqrt(jnp.sum(h * h, axis=1, keepdims=True)), EPS)
    d = jax.lax.dot_general(x, h, (((1,), (1,)), ((), ())),
                            preferred_element_type=jnp.float32)
    sims = d / (xn * hn.T)
    gidx = jax.lax.broadcasted_iota(jnp.int32, (B, KC), 1) + i * KC
    sims = jnp.where(gidx < K_HIST, sims, NEG)

    aug_v = jnp.concatenate([vals_s[...], sims], axis=1)
    aug_i = jnp.concatenate([idx_s[...], gidx], axis=1)

    ms, sels = [], []
    for _ in range(K_NEIGH):
        m = jnp.max(aug_v, axis=1, keepdims=True)
        sel = jnp.min(jnp.where(aug_v == m, aug_i, BIGI), axis=1, keepdims=True)
        aug_v = jnp.where(aug_i == sel, NEG, aug_v)
        ms.append(m)
        sels.append(sel)

    padv = jnp.full((B, 128 - K_NEIGH), NEG, jnp.float32)
    padi = jnp.full((B, 128 - K_NEIGH), BIGI, jnp.int32)
    vals_s[...] = jnp.concatenate(ms + [padv], axis=1)
    idx_s[...] = jnp.concatenate(sels + [padi], axis=1)

    @pl.when(i == NK - 1)
    def _emit():
        outv_ref[...] = jnp.concatenate(ms + [padv[:, :8 - K_NEIGH]], axis=1)
        outi_ref[...] = jnp.concatenate(sels + [padi[:, :8 - K_NEIGH]], axis=1)


def _topk(x_last, hist_pad):
    return pl.pallas_call(
        _topk_body,
        grid=(NK,),
        in_specs=[
            pl.BlockSpec((B, D), lambda i: (0, 0)),
            pl.BlockSpec((KC, D), lambda i: (i, 0)),
        ],
        out_specs=[
            pl.BlockSpec((B, 8), lambda i: (0, 0)),
            pl.BlockSpec((B, 8), lambda i: (0, 0)),
        ],
        out_shape=[
            jax.ShapeDtypeStruct((B, 8), jnp.float32),
            jax.ShapeDtypeStruct((B, 8), jnp.int32),
        ],
        scratch_shapes=[
            pltpu.VMEM((B, 128), jnp.float32),
            pltpu.VMEM((B, 128), jnp.int32),
        ],
    )(x_last, hist_pad)


def _sc_gather(table, idx_flat):
    info = plsc.get_sparse_core_info()
    nc, ns = info.num_cores, info.num_subcores
    nw = nc * ns
    nrows = idx_flat.shape[0]
    bpw = nrows // nw
    mesh = plsc.VectorSubcoreMesh(core_axis_name="c", subcore_axis_name="s")

    @functools.partial(
        pl.kernel,
        mesh=mesh,
        out_type=jax.ShapeDtypeStruct((nrows, D), jnp.float32),
        scratch_types=[
            pltpu.VMEM((bpw,), jnp.int32),
            pltpu.VMEM((bpw, D), jnp.float32),
            pltpu.SemaphoreType.DMA,
        ],
        compiler_params=pltpu.CompilerParams(use_tc_tiling_on_sc=False),
    )
    def k(table_hbm, idx_hbm, out_hbm, idx_v, rows_v, sem):
        wid = lax.axis_index("s") * nc + lax.axis_index("c")
        base = wid * bpw
        pltpu.sync_copy(idx_hbm.at[pl.ds(base, bpw)], idx_v)
        pltpu.async_copy(table_hbm.at[idx_v], rows_v, sem).wait()
        pltpu.sync_copy(rows_v, out_hbm.at[pl.ds(base, bpw)])

    return k(table, idx_flat)


def _fuse_body(v_ref, rows_ref, x_ref, w1a_ref, w1s_ref, b1_ref, w2_ref,
               b2_ref, g_ref, bt_ref, o_ref):
    v = v_ref[:, :K_NEIGH]
    m = jnp.max(v, axis=1, keepdims=True)
    e = jnp.exp(v - m)
    w = e / jnp.sum(e, axis=1, keepdims=True)
    rows = rows_ref[...]
    ws = jnp.zeros((B, D), jnp.float32)
    for j in range(K_NEIGH):
        ws = ws + rows[:, j * D:(j + 1) * D] * w[:, j:j + 1]
    x = x_ref[...]
    h = jax.lax.dot_general(x, w1a_ref[...], (((1,), (0,)), ((), ())),
                            preferred_element_type=jnp.float32)
    h = h + jax.lax.dot_general(ws, w1s_ref[...], (((1,), (0,)), ((), ())),
                                preferred_element_type=jnp.float32)
    h = jnp.maximum(h + b1_ref[...], 0.0)
    h = jax.lax.dot_general(h, w2_ref[...], (((1,), (0,)), ((), ())),
                            preferred_element_type=jnp.float32) + b2_ref[...]
    mu = jnp.mean(h, axis=-1, keepdims=True)
    var = jnp.mean((h - mu) ** 2, axis=-1, keepdims=True)
    o_ref[...] = (h - mu) / jnp.sqrt(var + 1e-5) * g_ref[...] + bt_ref[...]


def _fuse(top5v, rows_flat, x_last, w1a, w1s, b1, w2, b2, gamma, beta):
    return pl.pallas_call(
        _fuse_body,
        out_shape=jax.ShapeDtypeStruct((B, D), jnp.float32),
    )(top5v, rows_flat, x_last, w1a, w1s, b1[None, :], w2, b2[None, :],
      gamma[None, :], beta[None, :])


def kernel(x, historical_data, W1, b1, W2, b2, gamma, beta):
    x_last = x[:, -1, :]
    hist_pad = jnp.pad(historical_data, ((0, K_PAD - K_HIST), (0, 0)))
    top5v, top5i = _topk(x_last, hist_pad)
    idx_flat = top5i[:, :K_NEIGH].reshape(B * K_NEIGH)
    rows = _sc_gather(historical_data, idx_flat)
    rows_flat = rows.reshape(B, K_NEIGH * D)
    w1a = W1[:D, :]
    w1s = jnp.sum(W1[D:, :].reshape(K_NEIGH, D, 2 * D), axis=0)
    return _fuse(top5v, rows_flat, x_last, w1a, w1s, b1, W2, b2, gamma, beta)


# speedup vs baseline: 5.0440x; 2.0806x over previous
import functools

import jax
import jax.numpy as jnp
from jax import lax
from jax.experimental import pallas as pl
from jax.experimental.pallas import tpu as pltpu
from jax.experimental.pallas import tpu_sc as plsc

B = 1024
D = 16
K_HIST = 100000
K_NEIGH = 5
KC = 2048
K_PAD = 100352
NK = K_PAD // KC
GPC = KC // 128
NG = K_PAD // 128
NEG = -3.0e38
BIGI = 0x3FFFFFFF
EPS = 1e-8


def _sims_body(x_ref, h_ref, sims_ref, gmax_ref):
    i = pl.program_id(0)
    x = x_ref[...]
    xn = jnp.maximum(jnp.sqrt(jnp.sum(x * x, axis=1, keepdims=True)), EPS)
    h = h_ref[...]
    hn = jnp.maximum(jnp.sqrt(jnp.sum(h * h, axis=1, keepdims=True)), EPS)
    d = jax.lax.dot_general(x, h, (((1,), (1,)), ((), ())),
                            preferred_element_type=jnp.float32)
    sims = d / (xn * hn.T)
    ridx = jax.lax.broadcasted_iota(jnp.int32, (1, KC), 1) + i * KC
    sims = sims + jnp.where(ridx < K_HIST, 0.0, NEG)
    gm = []
    for g in range(GPC):
        blk = sims[:, g * 128:(g + 1) * 128]
        sims_ref[g, :, :] = blk
        gm.append(jnp.max(blk, axis=1, keepdims=True))
    gmax_ref[0, :, :] = jnp.concatenate(gm, axis=1)


def _sims(x_last, hist_pad):
    return pl.pallas_call(
        _sims_body,
        grid=(NK,),
        in_specs=[
            pl.BlockSpec((B, D), lambda i: (0, 0)),
            pl.BlockSpec((KC, D), lambda i: (i, 0)),
        ],
        out_specs=[
            pl.BlockSpec((GPC, B, 128), lambda i: (i, 0, 0)),
            pl.BlockSpec((1, B, GPC), lambda i: (i, 0, 0)),
        ],
        out_shape=[
            jax.ShapeDtypeStruct((NG, B, 128), jnp.float32),
            jax.ShapeDtypeStruct((NK, B, GPC), jnp.float32),
        ],
    )(x_last, hist_pad)


def _topgrp_body(gm_ref, grp_ref):
    v = gm_ref[...]
    gi = jax.lax.broadcasted_iota(jnp.int32, (B, NG), 1)
    sels = []
    for _ in range(K_NEIGH):
        m = jnp.max(v, axis=1, keepdims=True)
        sel = jnp.min(jnp.where(v == m, gi, BIGI), axis=1, keepdims=True)
        v = jnp.where(gi == sel, NEG, v)
        sels.append(sel)
    pad = jnp.full((B, 8 - K_NEIGH), 0, jnp.int32)
    grp_ref[...] = jnp.concatenate(sels + [pad], axis=1)


def _topgrp(gmax):
    return pl.pallas_call(
        _topgrp_body,
        out_shape=jax.ShapeDtypeStruct((B, 8), jnp.int32),
    )(gmax)


def _topel_body(segs_ref, grp_ref, outv_ref, outi_ref):
    il = jax.lax.broadcasted_iota(jnp.int32, (B, 128), 1)
    vs, idxs = [], []
    for j in range(K_NEIGH):
        vs.append(segs_ref[j, :, :])
        idxs.append(grp_ref[:, j:j + 1] * 128 + il)
    aug_v = jnp.concatenate(vs, axis=1)
    aug_i = jnp.concatenate(idxs, axis=1)
    ms, sels = [], []
    for _ in range(K_NEIGH):
        m = jnp.max(aug_v, axis=1, keepdims=True)
        sel = jnp.min(jnp.where(aug_v == m, aug_i, BIGI), axis=1, keepdims=True)
        aug_v = jnp.where(aug_i == sel, NEG, aug_v)
        ms.append(m)
        sels.append(sel)
    outv_ref[...] = jnp.concatenate(
        ms + [jnp.full((B, 8 - K_NEIGH), NEG, jnp.float32)], axis=1)
    outi_ref[...] = jnp.concatenate(
        sels + [jnp.full((B, 8 - K_NEIGH), BIGI, jnp.int32)], axis=1)


def _topel(segs, grp8):
    return pl.pallas_call(
        _topel_body,
        out_shape=[
            jax.ShapeDtypeStruct((B, 8), jnp.float32),
            jax.ShapeDtypeStruct((B, 8), jnp.int32),
        ],
    )(segs, grp8)


def _sc_gather(table, idx_flat, width):
    info = plsc.get_sparse_core_info()
    nc, ns = info.num_cores, info.num_subcores
    nw = nc * ns
    nrows = idx_flat.shape[0]
    bpw = nrows // nw
    mesh = plsc.VectorSubcoreMesh(core_axis_name="c", subcore_axis_name="s")

    @functools.partial(
        pl.kernel,
        mesh=mesh,
        out_type=jax.ShapeDtypeStruct((nrows, width), jnp.float32),
        scratch_types=[
            pltpu.VMEM((bpw,), jnp.int32),
            pltpu.VMEM((bpw, width), jnp.float32),
            pltpu.SemaphoreType.DMA,
        ],
        compiler_params=pltpu.CompilerParams(use_tc_tiling_on_sc=False),
    )
    def k(table_hbm, idx_hbm, out_hbm, idx_v, rows_v, sem):
        wid = lax.axis_index("s") * nc + lax.axis_index("c")
        base = wid * bpw
        pltpu.sync_copy(idx_hbm.at[pl.ds(base, bpw)], idx_v)
        pltpu.async_copy(table_hbm.at[idx_v], rows_v, sem).wait()
        pltpu.sync_copy(rows_v, out_hbm.at[pl.ds(base, bpw)])

    return k(table, idx_flat)


def _fuse_body(v_ref, rows_ref, x_ref, w1a_ref, w1s_ref, b1_ref, w2_ref,
               b2_ref, g_ref, bt_ref, o_ref):
    v = v_ref[:, :K_NEIGH]
    m = jnp.max(v, axis=1, keepdims=True)
    e = jnp.exp(v - m)
    w = e / jnp.sum(e, axis=1, keepdims=True)
    rows = rows_ref[...]
    ws = jnp.zeros((B, D), jnp.float32)
    for j in range(K_NEIGH):
        ws = ws + rows[:, j * D:(j + 1) * D] * w[:, j:j + 1]
    x = x_ref[...]
    h = jax.lax.dot_general(x, w1a_ref[...], (((1,), (0,)), ((), ())),
                            preferred_element_type=jnp.float32)
    h = h + jax.lax.dot_general(ws, w1s_ref[...], (((1,), (0,)), ((), ())),
                                preferred_element_type=jnp.float32)
    h = jnp.maximum(h + b1_ref[...], 0.0)
    h = jax.lax.dot_general(h, w2_ref[...], (((1,), (0,)), ((), ())),
                            preferred_element_type=jnp.float32) + b2_ref[...]
    mu = jnp.mean(h, axis=-1, keepdims=True)
    var = jnp.mean((h - mu) ** 2, axis=-1, keepdims=True)
    o_ref[...] = (h - mu) / jnp.sqrt(var + 1e-5) * g_ref[...] + bt_ref[...]


def _fuse(top5v, rows_flat, x_last, w1a, w1s, b1, w2, b2, gamma, beta):
    return pl.pallas_call(
        _fuse_body,
        out_shape=jax.ShapeDtypeStruct((B, D), jnp.float32),
    )(top5v, rows_flat, x_last, w1a, w1s, b1[None, :], w2, b2[None, :],
      gamma[None, :], beta[None, :])


def kernel(x, historical_data, W1, b1, W2, b2, gamma, beta):
    x_last = x[:, -1, :]
    hist_pad = jnp.pad(historical_data, ((0, K_PAD - K_HIST), (0, 0)))
    sims_t, gmax3 = _sims(x_last, hist_pad)
    gmax = gmax3.transpose(1, 0, 2).reshape(B, NG)
    grp8 = _topgrp(gmax)
    seg_fl = (grp8[:, :K_NEIGH] * B
              + jnp.arange(B, dtype=jnp.int32)[:, None]).T.reshape(B * K_NEIGH)
    segs = _sc_gather(sims_t.reshape(NG * B, 128), seg_fl, 128)
    segs = segs.reshape(K_NEIGH, B, 128)
    top5v, top5i = _topel(segs, grp8)
    idx_flat = top5i[:, :K_NEIGH].reshape(B * K_NEIGH)
    rows = _sc_gather(historical_data, idx_flat, D)
    rows_flat = rows.reshape(B, K_NEIGH * D)
    w1a = W1[:D, :]
    w1s = jnp.sum(W1[D:, :].reshape(K_NEIGH, D, 2 * D), axis=0)
    return _fuse(top5v, rows_flat, x_last, w1a, w1s, b1, W2, b2, gamma, beta)


# speedup vs baseline: 5.7928x; 1.1485x over previous
import functools

import jax
import jax.numpy as jnp
from jax import lax
from jax.experimental import pallas as pl
from jax.experimental.pallas import tpu as pltpu
from jax.experimental.pallas import tpu_sc as plsc

B = 1024
D = 16
K_HIST = 100000
K_NEIGH = 5
KC = 2048
K_PAD = 100352
NK = K_PAD // KC
GPC = KC // 128
NG = K_PAD // 128
LPC = KC // 8
NEG = -3.0e38
BIGI = 0x3FFFFFFF
EPS = 1e-8


def _sims_body(h_ref, x_ref, sims_ref, gmax_ref):
    i = pl.program_id(0)
    h = h_ref[...]
    hn = jnp.maximum(jnp.sqrt(jnp.sum(h * h, axis=1, keepdims=True)), EPS)
    inv_h = 1.0 / hn
    x = x_ref[...]
    d = jax.lax.dot_general(x, h, (((1,), (1,)), ((), ())),
                            preferred_element_type=jnp.float32)
    ridx = jax.lax.broadcasted_iota(jnp.int32, (1, KC), 1) + i * KC
    sims = jnp.where(ridx < K_HIST, d * inv_h.T, NEG)
    gm = []
    for g in range(GPC):
        blk = sims[:, g * 128:(g + 1) * 128]
        sims_ref[g, :, :] = blk
        gm.append(jnp.max(blk, axis=1, keepdims=True))
    gmax_ref[0, :, :] = jnp.concatenate(gm, axis=1)


def _sims(x_last, hist):
    return pl.pallas_call(
        _sims_body,
        grid=(NK,),
        in_specs=[
            pl.BlockSpec((KC, D), lambda i: (i, 0)),
            pl.BlockSpec((B, D), lambda i: (0, 0)),
        ],
        out_specs=[
            pl.BlockSpec((GPC, B, 128), lambda i: (i, 0, 0)),
            pl.BlockSpec((1, B, GPC), lambda i: (i, 0, 0)),
        ],
        out_shape=[
            jax.ShapeDtypeStruct((NG, B, 128), jnp.float32),
            jax.ShapeDtypeStruct((NK, B, GPC), jnp.float32),
        ],
    )(hist, x_last)


def _topgrp_body(gm_ref, grp_ref):
    v = gm_ref[...]
    gi = jax.lax.broadcasted_iota(jnp.int32, (B, NG), 1)
    sels = []
    for _ in range(K_NEIGH):
        m = jnp.max(v, axis=1, keepdims=True)
        sel = jnp.min(jnp.where(v == m, gi, BIGI), axis=1, keepdims=True)
        v = jnp.where(gi == sel, NEG, v)
        sels.append(sel)
    pad = jnp.full((B, 8 - K_NEIGH), 0, jnp.int32)
    grp_ref[...] = jnp.concatenate(sels + [pad], axis=1)


def _topgrp(gmax):
    return pl.pallas_call(
        _topgrp_body,
        out_shape=jax.ShapeDtypeStruct((B, 8), jnp.int32),
    )(gmax)


def _topel_body(segs_ref, grp_ref, x_ref, outv_ref, outi_ref):
    il = jax.lax.broadcasted_iota(jnp.int32, (B, 128), 1)
    vs, idxs = [], []
    for j in range(K_NEIGH):
        vs.append(segs_ref[j, :, :])
        idxs.append(grp_ref[:, j:j + 1] * 128 + il)
    aug_v = jnp.concatenate(vs, axis=1)
    aug_i = jnp.concatenate(idxs, axis=1)
    ms, sels = [], []
    for _ in range(K_NEIGH):
        m = jnp.max(aug_v, axis=1, keepdims=True)
        sel = jnp.min(jnp.where(aug_v == m, aug_i, BIGI), axis=1, keepdims=True)
        aug_v = jnp.where(aug_i == sel, NEG, aug_v)
        ms.append(m)
        sels.append(sel)
    x = x_ref[...]
    inv_x = 1.0 / jnp.maximum(
        jnp.sqrt(jnp.sum(x * x, axis=1, keepdims=True)), EPS)
    outv_ref[...] = jnp.concatenate(
        [m * inv_x for m in ms]
        + [jnp.full((B, 8 - K_NEIGH), NEG, jnp.float32)], axis=1)
    outi_ref[...] = jnp.concatenate(
        sels + [jnp.full((B, 8 - K_NEIGH), BIGI, jnp.int32)], axis=1)


def _topel(segs, grp8, x_last):
    return pl.pallas_call(
        _topel_body,
        out_shape=[
            jax.ShapeDtypeStruct((B, 8), jnp.float32),
            jax.ShapeDtypeStruct((B, 8), jnp.int32),
        ],
    )(segs, grp8, x_last)


def _sc_gather(table, idx_flat, width):
    info = plsc.get_sparse_core_info()
    nc, ns = info.num_cores, info.num_subcores
    nw = nc * ns
    nrows = idx_flat.shape[0]
    bpw = nrows // nw
    mesh = plsc.VectorSubcoreMesh(core_axis_name="c", subcore_axis_name="s")

    @functools.partial(
        pl.kernel,
        mesh=mesh,
        out_type=jax.ShapeDtypeStruct((nrows, width), jnp.float32),
        scratch_types=[
            pltpu.VMEM((bpw,), jnp.int32),
            pltpu.VMEM((bpw, width), jnp.float32),
            pltpu.SemaphoreType.DMA,
        ],
        compiler_params=pltpu.CompilerParams(use_tc_tiling_on_sc=False),
    )
    def k(table_hbm, idx_hbm, out_hbm, idx_v, rows_v, sem):
        wid = lax.axis_index("s") * nc + lax.axis_index("c")
        base = wid * bpw
        pltpu.sync_copy(idx_hbm.at[pl.ds(base, bpw)], idx_v)
        pltpu.async_copy(table_hbm.at[idx_v], rows_v, sem).wait()
        pltpu.sync_copy(rows_v, out_hbm.at[pl.ds(base, bpw)])

    return k(table, idx_flat)


def _fuse_body(v_ref, rows_ref, x_ref, w1a_ref, w1s_ref, b1_ref, w2_ref,
               b2_ref, g_ref, bt_ref, o_ref):
    v = v_ref[:, :K_NEIGH]
    m = jnp.max(v, axis=1, keepdims=True)
    e = jnp.exp(v - m)
    w = e / jnp.sum(e, axis=1, keepdims=True)
    rows = rows_ref[...]
    ws = jnp.zeros((B, D), jnp.float32)
    for j in range(K_NEIGH):
        ws = ws + rows[:, j * D:(j + 1) * D] * w[:, j:j + 1]
    x = x_ref[...]
    h = jax.lax.dot_general(x, w1a_ref[...], (((1,), (0,)), ((), ())),
                            preferred_element_type=jnp.float32)
    h = h + jax.lax.dot_general(ws, w1s_ref[...], (((1,), (0,)), ((), ())),
                                preferred_element_type=jnp.float32)
    h = jnp.maximum(h + b1_ref[...], 0.0)
    h = jax.lax.dot_general(h, w2_ref[...], (((1,), (0,)), ((), ())),
                            preferred_element_type=jnp.float32) + b2_ref[...]
    mu = jnp.mean(h, axis=-1, keepdims=True)
    var = jnp.mean((h - mu) ** 2, axis=-1, keepdims=True)
    o_ref[...] = (h - mu) / jnp.sqrt(var + 1e-5) * g_ref[...] + bt_ref[...]


def _fuse(top5v, rows_flat, x_last, w1a, w1s, b1, w2, b2, gamma, beta):
    return pl.pallas_call(
        _fuse_body,
        out_shape=jax.ShapeDtypeStruct((B, D), jnp.float32),
    )(top5v, rows_flat, x_last, w1a, w1s, b1[None, :], w2, b2[None, :],
      gamma[None, :], beta[None, :])


def kernel(x, historical_data, W1, b1, W2, b2, gamma, beta):
    x_last = x[:, -1, :]
    sims_t, gmax3 = _sims(x_last, historical_data)
    gmax = gmax3.transpose(1, 0, 2).reshape(B, NG)
    grp8 = _topgrp(gmax)
    seg_fl = (grp8[:, :K_NEIGH] * B
              + jnp.arange(B, dtype=jnp.int32)[:, None]).T.reshape(B * K_NEIGH)
    segs = _sc_gather(sims_t.reshape(NG * B, 128), seg_fl, 128)
    segs = segs.reshape(K_NEIGH, B, 128)
    top5v, top5i = _topel(segs, grp8, x_last)
    idx_flat = top5i[:, :K_NEIGH].reshape(B * K_NEIGH)
    rows = _sc_gather(historical_data, idx_flat, D)
    rows_flat = rows.reshape(B, K_NEIGH * D)
    w1a = W1[:D, :]
    w1s = jnp.sum(W1[D:, :].reshape(K_NEIGH, D, 2 * D), axis=0)
    return _fuse(top5v, rows_flat, x_last, w1a, w1s, b1, W2, b2, gamma, beta)


# speedup vs baseline: 5.8709x; 1.0135x over previous
import functools

import jax
import jax.numpy as jnp
from jax import lax
from jax.experimental import pallas as pl
from jax.experimental.pallas import tpu as pltpu
from jax.experimental.pallas import tpu_sc as plsc

B = 1024
D = 16
K_HIST = 100000
K_NEIGH = 5
KC = 2048
K_PAD = 100352
NK = K_PAD // KC
GPC = KC // 128
NG = K_PAD // 128
LPC = KC // 8
NEG = -3.0e38
BIGI = 0x3FFFFFFF
EPS = 1e-8


def _sims_body(h_ref, x_ref, sims_ref, gmax_ref):
    i = pl.program_id(0)
    h = h_ref[...]
    hn = jnp.maximum(jnp.sqrt(jnp.sum(h * h, axis=1, keepdims=True)), EPS)
    inv_h = 1.0 / hn
    x = x_ref[...]
    d = jax.lax.dot_general(x, h, (((1,), (1,)), ((), ())),
                            preferred_element_type=jnp.float32)
    ridx = jax.lax.broadcasted_iota(jnp.int32, (1, KC), 1) + i * KC
    sims = jnp.where(ridx < K_HIST, d * inv_h.T, NEG)
    gm = []
    for g in range(GPC):
        blk = sims[:, g * 128:(g + 1) * 128]
        sims_ref[g, :, :] = blk
        gm.append(jnp.max(blk, axis=1, keepdims=True))
    gmax_ref[0, :, :] = jnp.concatenate(gm, axis=1)


def _sims(x_last, hist):
    return pl.pallas_call(
        _sims_body,
        grid=(NK,),
        in_specs=[
            pl.BlockSpec((KC, D), lambda i: (i, 0)),
            pl.BlockSpec((B, D), lambda i: (0, 0)),
        ],
        out_specs=[
            pl.BlockSpec((GPC, B, 128), lambda i: (i, 0, 0)),
            pl.BlockSpec((1, B, GPC), lambda i: (i, 0, 0)),
        ],
        out_shape=[
            jax.ShapeDtypeStruct((NG, B, 128), jnp.float32),
            jax.ShapeDtypeStruct((NK, B, GPC), jnp.float32),
        ],
    )(hist, x_last)


def _topgrp_body(gm_ref, grp_ref):
    v = jnp.concatenate([gm_ref[c, :, :] for c in range(NK)], axis=1)
    gi = jax.lax.broadcasted_iota(jnp.int32, (B, NG), 1)
    sels = []
    for _ in range(K_NEIGH):
        m = jnp.max(v, axis=1, keepdims=True)
        sel = jnp.min(jnp.where(v == m, gi, BIGI), axis=1, keepdims=True)
        v = jnp.where(gi == sel, NEG, v)
        sels.append(sel)
    pad = jnp.full((B, 8 - K_NEIGH), 0, jnp.int32)
    grp_ref[...] = jnp.concatenate(sels + [pad], axis=1)


def _topgrp(gmax):
    return pl.pallas_call(
        _topgrp_body,
        out_shape=jax.ShapeDtypeStruct((B, 8), jnp.int32),
    )(gmax)


def _topel_body(segs_ref, grp_ref, x_ref, outv_ref, outi_ref):
    il = jax.lax.broadcasted_iota(jnp.int32, (B, 128), 1)
    vs, idxs = [], []
    for j in range(K_NEIGH):
        vs.append(segs_ref[j, :, :])
        idxs.append(grp_ref[:, j:j + 1] * 128 + il)
    aug_v = jnp.concatenate(vs, axis=1)
    aug_i = jnp.concatenate(idxs, axis=1)
    ms, sels = [], []
    for _ in range(K_NEIGH):
        m = jnp.max(aug_v, axis=1, keepdims=True)
        sel = jnp.min(jnp.where(aug_v == m, aug_i, BIGI), axis=1, keepdims=True)
        aug_v = jnp.where(aug_i == sel, NEG, aug_v)
        ms.append(m)
        sels.append(sel)
    x = x_ref[...]
    inv_x = 1.0 / jnp.maximum(
        jnp.sqrt(jnp.sum(x * x, axis=1, keepdims=True)), EPS)
    outv_ref[...] = jnp.concatenate(
        [m * inv_x for m in ms]
        + [jnp.full((B, 8 - K_NEIGH), NEG, jnp.float32)], axis=1)
    outi_ref[...] = jnp.concatenate(
        sels + [jnp.full((B, 8 - K_NEIGH), BIGI, jnp.int32)], axis=1)


def _topel(segs, grp8, x_last):
    return pl.pallas_call(
        _topel_body,
        out_shape=[
            jax.ShapeDtypeStruct((B, 8), jnp.float32),
            jax.ShapeDtypeStruct((B, 8), jnp.int32),
        ],
    )(segs, grp8, x_last)


def _sc_gather(table, idx_flat, width):
    info = plsc.get_sparse_core_info()
    nc, ns = info.num_cores, info.num_subcores
    nw = nc * ns
    nrows = idx_flat.shape[0]
    bpw = nrows // nw
    mesh = plsc.VectorSubcoreMesh(core_axis_name="c", subcore_axis_name="s")

    @functools.partial(
        pl.kernel,
        mesh=mesh,
        out_type=jax.ShapeDtypeStruct((nrows, width), jnp.float32),
        scratch_types=[
            pltpu.VMEM((bpw,), jnp.int32),
            pltpu.VMEM((bpw, width), jnp.float32),
            pltpu.SemaphoreType.DMA,
        ],
        compiler_params=pltpu.CompilerParams(use_tc_tiling_on_sc=False),
    )
    def k(table_hbm, idx_hbm, out_hbm, idx_v, rows_v, sem):
        wid = lax.axis_index("s") * nc + lax.axis_index("c")
        base = wid * bpw
        pltpu.sync_copy(idx_hbm.at[pl.ds(base, bpw)], idx_v)
        pltpu.async_copy(table_hbm.at[idx_v], rows_v, sem).wait()
        pltpu.sync_copy(rows_v, out_hbm.at[pl.ds(base, bpw)])

    return k(table, idx_flat)


def _fuse_body(v_ref, rows_ref, x_ref, w1a_ref, w1s_ref, b1_ref, w2_ref,
               b2_ref, g_ref, bt_ref, o_ref):
    v = v_ref[:, :K_NEIGH]
    m = jnp.max(v, axis=1, keepdims=True)
    e = jnp.exp(v - m)
    w = e / jnp.sum(e, axis=1, keepdims=True)
    rows = rows_ref[...]
    ws = jnp.zeros((B, D), jnp.float32)
    for j in range(K_NEIGH):
        ws = ws + rows[:, j * D:(j + 1) * D] * w[:, j:j + 1]
    x = x_ref[...]
    h = jax.lax.dot_general(x, w1a_ref[...], (((1,), (0,)), ((), ())),
                            preferred_element_type=jnp.float32)
    h = h + jax.lax.dot_general(ws, w1s_ref[...], (((1,), (0,)), ((), ())),
                                preferred_element_type=jnp.float32)
    h = jnp.maximum(h + b1_ref[...], 0.0)
    h = jax.lax.dot_general(h, w2_ref[...], (((1,), (0,)), ((), ())),
                            preferred_element_type=jnp.float32) + b2_ref[...]
    mu = jnp.mean(h, axis=-1, keepdims=True)
    var = jnp.mean((h - mu) ** 2, axis=-1, keepdims=True)
    o_ref[...] = (h - mu) / jnp.sqrt(var + 1e-5) * g_ref[...] + bt_ref[...]


def _fuse(top5v, rows_flat, x_last, w1a, w1s, b1, w2, b2, gamma, beta):
    return pl.pallas_call(
        _fuse_body,
        out_shape=jax.ShapeDtypeStruct((B, D), jnp.float32),
    )(top5v, rows_flat, x_last, w1a, w1s, b1[None, :], w2, b2[None, :],
      gamma[None, :], beta[None, :])


def kernel(x, historical_data, W1, b1, W2, b2, gamma, beta):
    x_last = x[:, -1, :]
    sims_t, gmax3 = _sims(x_last, historical_data)
    grp8 = _topgrp(gmax3)
    seg_fl = (grp8[:, :K_NEIGH] * B
              + jnp.arange(B, dtype=jnp.int32)[:, None]).T.reshape(B * K_NEIGH)
    segs = _sc_gather(sims_t.reshape(NG * B, 128), seg_fl, 128)
    segs = segs.reshape(K_NEIGH, B, 128)
    top5v, top5i = _topel(segs, grp8, x_last)
    idx_flat = top5i[:, :K_NEIGH].reshape(B * K_NEIGH)
    rows = _sc_gather(historical_data, idx_flat, D)
    rows_flat = rows.reshape(B, K_NEIGH * D)
    w1a = W1[:D, :]
    w1s = jnp.sum(W1[D:, :].reshape(K_NEIGH, D, 2 * D), axis=0)
    return _fuse(top5v, rows_flat, x_last, w1a, w1s, b1, W2, b2, gamma, beta)
